# baseline (speedup 1.0000x reference)
import functools

import jax
import jax.numpy as jnp
from jax import lax
from jax.experimental import pallas as pl
from jax.experimental.pallas import tpu as pltpu
from jax.experimental.pallas import tpu_sc as plsc

Z = 32
B = 16384

_NC = 2
_NS = 16
_NW = _NC * _NS
_BPW = B // _NW


def _make_kernel():
    mesh = plsc.VectorSubcoreMesh(core_axis_name="c", subcore_axis_name="s")

    @functools.partial(
        pl.kernel,
        mesh=mesh,
        compiler_params=pltpu.CompilerParams(use_tc_tiling_on_sc=False),
        out_type=(
            jax.ShapeDtypeStruct((B, Z), jnp.float32),
            jax.ShapeDtypeStruct((B, Z), jnp.float32),
        ),
        scratch_types=[
            pltpu.VMEM((_BPW,), jnp.int32),
            pltpu.VMEM((_BPW, Z), jnp.float32),
            pltpu.VMEM((_BPW, Z), jnp.float32),
            pltpu.SemaphoreType.DMA,
            pltpu.SemaphoreType.DMA,
        ],
    )
    def k(u_hbm, mean_hbm, logvar_hbm, mean_out, logvar_out,
          idx_v, mrows_v, lrows_v, sem_m, sem_l):
        wid = lax.axis_index("s") * _NC + lax.axis_index("c")
        base = wid * _BPW
        pltpu.sync_copy(u_hbm.at[pl.ds(base, _BPW)], idx_v)
        cm = pltpu.async_copy(mean_hbm.at[idx_v], mrows_v, sem_m)
        cl = pltpu.async_copy(logvar_hbm.at[idx_v], lrows_v, sem_l)
        cm.wait()
        pltpu.sync_copy(mrows_v, mean_out.at[pl.ds(base, _BPW)])
        cl.wait()
        pltpu.sync_copy(lrows_v, logvar_out.at[pl.ds(base, _BPW)])

    return k


_gather2 = jax.jit(_make_kernel())


def kernel(u, mean_table, log_variance_table):
    return _gather2(u, mean_table, log_variance_table)
